# baseline (speedup 1.0000x reference)
import functools

import jax
import jax.numpy as jnp
from jax import lax
from jax.experimental import pallas as pl
from jax.experimental.pallas import tpu as pltpu
from jax.experimental.pallas import tpu_sc as plsc

N = 10000
E = 320000
D = 128
H1 = 8
C1 = 8
HC = 64
NCLS = 40
W = HC + 16

NC = 2
NS = 16
NW = NC * NS
EPW = E // NW
CH = 125
EPC = EPW // CH
NP = 12000
STRIPE = NP // NS

_BN = 2000

_SC_PARAMS = pltpu.CompilerParams(use_tc_tiling_on_sc=False)

_NEG = -1e30


def _lrelu(x):
    return jnp.where(x >= 0, x, 0.2 * x)


def _prep1_body(x_ref, w1_ref, asrc_ref, adst_ref, xpp_ref, tsl_ref, td_ref,
                m1_ref, mx_ref):
    i = pl.program_id(0)
    ng = pl.num_programs(0)
    xp = jnp.dot(x_ref[...], w1_ref[...], preferred_element_type=jnp.float32)
    xph = xp.reshape(_BN, H1, C1)
    xpp_ref[...] = jnp.transpose(xph, (0, 2, 1)).reshape(_BN, HC)
    a_s = jnp.sum(xph * asrc_ref[...][None, :, :], axis=-1)
    a_d = jnp.sum(xph * adst_ref[...][None, :, :], axis=-1)
    tsl_ref[...] = jnp.concatenate([a_s, a_s], axis=1)
    td_ref[...] = jnp.concatenate([a_d, a_d], axis=1)
    blkmax = jnp.concatenate(
        [jnp.max(a_s, axis=0), jnp.max(a_d, axis=0)])[None, :]

    @pl.when(i == 0)
    def _():
        mx_ref[...] = blkmax

    @pl.when(i > 0)
    def _():
        mx_ref[...] = jnp.maximum(mx_ref[...], blkmax)

    @pl.when(i == ng - 1)
    def _():
        mx = mx_ref[...]
        m = _lrelu(mx[:, 0:H1] + mx[:, H1:2 * H1])
        m1_ref[...] = jnp.concatenate([m, m], axis=1)


def _prep1(x, W1, att_src1, att_dst1):
    return pl.pallas_call(
        _prep1_body,
        grid=(N // _BN,),
        in_specs=[
            pl.BlockSpec((_BN, D), lambda i: (i, 0)),
            pl.BlockSpec((D, HC), lambda i: (0, 0)),
            pl.BlockSpec((H1, C1), lambda i: (0, 0)),
            pl.BlockSpec((H1, C1), lambda i: (0, 0)),
        ],
        out_specs=[
            pl.BlockSpec((_BN, HC), lambda i: (i, 0)),
            pl.BlockSpec((_BN, 16), lambda i: (i, 0)),
            pl.BlockSpec((_BN, 16), lambda i: (i, 0)),
            pl.BlockSpec((1, 16), lambda i: (0, 0)),
        ],
        out_shape=[
            jax.ShapeDtypeStruct((N, HC), jnp.float32),
            jax.ShapeDtypeStruct((N, 16), jnp.float32),
            jax.ShapeDtypeStruct((N, 16), jnp.float32),
            jax.ShapeDtypeStruct((1, 16), jnp.float32),
        ],
        scratch_shapes=[pltpu.VMEM((1, 16), jnp.float32)],
    )(x, W1, att_src1, att_dst1)


def _sc_layer(xpp, tsl, td, src3, dst3, m, z64, z16):
    mesh = plsc.VectorSubcoreMesh(core_axis_name="c", subcore_axis_name="s")

    @functools.partial(
        pl.kernel,
        out_type=(
            jax.ShapeDtypeStruct((NC, NP, HC), jnp.float32),
            jax.ShapeDtypeStruct((NC, NP, 16), jnp.float32),
        ),
        mesh=mesh,
        compiler_params=_SC_PARAMS,
        scratch_types=[
            pltpu.VMEM((EPC, CH), jnp.int32),
            pltpu.VMEM((EPC, CH), jnp.int32),
            pltpu.VMEM((16,), jnp.float32),
            pltpu.VMEM((2, CH, HC), jnp.float32),
            pltpu.VMEM((2, CH, 16), jnp.float32),
            pltpu.VMEM((2, CH, 16), jnp.float32),
            pltpu.VMEM((2, CH, HC), jnp.float32),
            pltpu.VMEM((2, CH, 16), jnp.float32),
            pltpu.VMEM_SHARED((NP, HC), jnp.float32),
            pltpu.VMEM_SHARED((NP, 16), jnp.float32),
            pltpu.SemaphoreType.DMA((2,)),
            pltpu.SemaphoreType.DMA((2,)),
        ],
    )
    def k(xpp_hbm, tsl_hbm, td_hbm, src_hbm, dst_hbm, m_hbm, z64_hbm,
          z16_hbm, accm_out, accd_out, src_v, dst_v, m_v, sb, lb, db,
          msgb, exb, accm, accd, gsem, ssem):
        c = lax.axis_index("c")
        s = lax.axis_index("s")
        wid = s * NC + c
        row0 = s * STRIPE
        pltpu.sync_copy(z64_hbm, accm.at[pl.ds(row0, STRIPE)])
        pltpu.sync_copy(z16_hbm, accd.at[pl.ds(row0, STRIPE)])
        pltpu.sync_copy(src_hbm.at[wid], src_v)
        pltpu.sync_copy(dst_hbm.at[wid], dst_v)
        pltpu.sync_copy(m_hbm, m_v)
        plsc.subcore_barrier()

        def fire(j, b):
            pltpu.make_async_copy(
                xpp_hbm.at[src_v.at[j]], sb.at[b], gsem.at[b]).start()
            pltpu.make_async_copy(
                tsl_hbm.at[src_v.at[j]], lb.at[b], gsem.at[b]).start()
            pltpu.make_async_copy(
                td_hbm.at[dst_v.at[j]], db.at[b], gsem.at[b]).start()

        def wait_g(b):
            pltpu.make_async_copy(
                xpp_hbm.at[src_v.at[0]], sb.at[b], gsem.at[b]).wait()
            pltpu.make_async_copy(
                tsl_hbm.at[src_v.at[0]], lb.at[b], gsem.at[b]).wait()
            pltpu.make_async_copy(
                td_hbm.at[src_v.at[0]], db.at[b], gsem.at[b]).wait()

        def work(j, b):
            mv = m_v[...]
            sbb = sb.at[b]
            lbb = lb.at[b]
            dbb = db.at[b]
            mbb = msgb.at[b]
            ebb = exb.at[b]

            @pl.loop(0, CH)
            def _(e):
                z = lbb[e, :] + dbb[e, :]
                ex = jnp.exp(_lrelu(z) - mv)
                ebb[e, :] = ex
                for g in range(4):
                    mbb[e, pl.ds(16 * g, 16)] = sbb[e, pl.ds(16 * g, 16)] * ex

            pltpu.async_copy(mbb, accm.at[dst_v.at[j]], ssem.at[b], add=True)
            pltpu.async_copy(ebb, accd.at[dst_v.at[j]], ssem.at[b], add=True)

        def wait_s(b):
            pltpu.make_async_copy(
                msgb.at[b], accm.at[dst_v.at[0]], ssem.at[b]).wait()
            pltpu.make_async_copy(
                exb.at[b], accd.at[dst_v.at[0]], ssem.at[b]).wait()

        fire(0, 0)

        @pl.loop(0, EPC, step=2)
        def _(j):
            @pl.when(j + 1 < EPC)
            def _():
                fire(j + 1, 1)
            wait_g(0)

            @pl.when(j >= 2)
            def _():
                wait_s(0)
            work(j, 0)

            @pl.when(j + 2 < EPC)
            def _():
                fire(j + 2, 0)
            wait_g(1)

            @pl.when(j >= 2)
            def _():
                wait_s(1)
            work(j + 1, 1)

        wait_s(0)
        wait_s(1)
        plsc.subcore_barrier()
        pltpu.sync_copy(accm.at[pl.ds(row0, STRIPE)],
                        accm_out.at[c, pl.ds(row0, STRIPE)])
        pltpu.sync_copy(accd.at[pl.ds(row0, STRIPE)],
                        accd_out.at[c, pl.ds(row0, STRIPE)])

    return k(xpp, tsl, td, src3, dst3, m, z64, z16)


def _ln(h, w, b):
    mu = jnp.mean(h, axis=-1, keepdims=True)
    var = jnp.mean((h - mu) ** 2, axis=-1, keepdims=True)
    return (h - mu) / jnp.sqrt(var + 1e-05) * w + b


def _post1_body(accm_ref, accd_ref, b1_ref, fc1w_ref, fc1b_ref, fc2w_ref,
                fc2b_ref, ln1w_ref, ln1b_ref, w2_ref, as2_ref, ad2_ref,
                h1_ref, xp2_ref, tsl2_ref, td2_ref, m2_ref, mx_ref):
    i = pl.program_id(0)
    ng = pl.num_programs(0)
    num = accm_ref[0] + accm_ref[1]
    den = accd_ref[0][:, 0:H1] + accd_ref[1][:, 0:H1]
    nperm = num.reshape(_BN, C1, H1)
    g = jnp.transpose(nperm, (0, 2, 1)) / (den[:, :, None] + 1e-16)
    h = jax.nn.relu(g.reshape(_BN, HC) + b1_ref[...])
    hc = h
    h = jax.nn.relu(
        jnp.dot(h, fc1w_ref[...], preferred_element_type=jnp.float32)
        + fc1b_ref[...])
    h = jax.nn.relu(
        jnp.dot(h, fc2w_ref[...], preferred_element_type=jnp.float32)
        + fc2b_ref[...] + hc)
    h = _ln(h, ln1w_ref[...], ln1b_ref[...])
    h1_ref[...] = h
    xp2 = jnp.dot(h, w2_ref[...], preferred_element_type=jnp.float32)
    xp2_ref[...] = xp2
    a_s2 = jnp.sum(xp2 * as2_ref[...], axis=1, keepdims=True)
    a_d2 = jnp.sum(xp2 * ad2_ref[...], axis=1, keepdims=True)
    tsl2_ref[...] = jnp.broadcast_to(a_s2, (_BN, 16))
    td2_ref[...] = jnp.broadcast_to(a_d2, (_BN, 16))
    blkmax = jnp.concatenate(
        [jnp.max(a_s2, axis=0), jnp.max(a_d2, axis=0),
         jnp.zeros((14,), jnp.float32)])[None, :]

    @pl.when(i == 0)
    def _():
        mx_ref[...] = blkmax

    @pl.when(i > 0)
    def _():
        mx_ref[...] = jnp.maximum(mx_ref[...], blkmax)

    @pl.when(i == ng - 1)
    def _():
        mx = mx_ref[...]
        m = _lrelu(mx[:, 0:1] + mx[:, 1:2])
        m2_ref[...] = jnp.broadcast_to(m, (1, 16))


def _post1(accm, accd, b1, fc1_w, fc1_b, fc2_w, fc2_b, ln1_w, ln1_b, W2,
           att_src2, att_dst2):
    return pl.pallas_call(
        _post1_body,
        grid=(N // _BN,),
        in_specs=[
            pl.BlockSpec((NC, _BN, HC), lambda i: (0, i, 0)),
            pl.BlockSpec((NC, _BN, 16), lambda i: (0, i, 0)),
            pl.BlockSpec((1, HC), lambda i: (0, 0)),
            pl.BlockSpec((HC, HC), lambda i: (0, 0)),
            pl.BlockSpec((1, HC), lambda i: (0, 0)),
            pl.BlockSpec((HC, HC), lambda i: (0, 0)),
            pl.BlockSpec((1, HC), lambda i: (0, 0)),
            pl.BlockSpec((1, HC), lambda i: (0, 0)),
            pl.BlockSpec((1, HC), lambda i: (0, 0)),
            pl.BlockSpec((HC, HC), lambda i: (0, 0)),
            pl.BlockSpec((1, HC), lambda i: (0, 0)),
            pl.BlockSpec((1, HC), lambda i: (0, 0)),
        ],
        out_specs=[
            pl.BlockSpec((_BN, HC), lambda i: (i, 0)),
            pl.BlockSpec((_BN, HC), lambda i: (i, 0)),
            pl.BlockSpec((_BN, 16), lambda i: (i, 0)),
            pl.BlockSpec((_BN, 16), lambda i: (i, 0)),
            pl.BlockSpec((1, 16), lambda i: (0, 0)),
        ],
        out_shape=[
            jax.ShapeDtypeStruct((N, HC), jnp.float32),
            jax.ShapeDtypeStruct((N, HC), jnp.float32),
            jax.ShapeDtypeStruct((N, 16), jnp.float32),
            jax.ShapeDtypeStruct((N, 16), jnp.float32),
            jax.ShapeDtypeStruct((1, 16), jnp.float32),
        ],
        scratch_shapes=[pltpu.VMEM((1, 16), jnp.float32)],
    )(accm, accd, b1, fc1_w, fc1_b, fc2_w, fc2_b, ln1_w, ln1_b, W2,
      att_src2, att_dst2)


def _post2_body(accm_ref, accd_ref, h1_ref, b2_ref, fc3w_ref, fc3b_ref,
                fc4w_ref, fc4b_ref, ln2w_ref, ln2b_ref, fcfw_ref, fcfb_ref,
                out_ref):
    num = accm_ref[0] + accm_ref[1]
    den = accd_ref[0][:, 0:1] + accd_ref[1][:, 0:1]
    h = num / (den + 1e-16) + b2_ref[...]
    h = jax.nn.relu(
        jnp.dot(h, fc3w_ref[...], preferred_element_type=jnp.float32)
        + fc3b_ref[...])
    h = (jnp.dot(h, fc4w_ref[...], preferred_element_type=jnp.float32)
         + fc4b_ref[...] + h1_ref[...])
    h = _ln(h, ln2w_ref[...], ln2b_ref[...])
    logits = (jnp.dot(h, fcfw_ref[...], preferred_element_type=jnp.float32)
              + fcfb_ref[...])
    m = jnp.max(logits, axis=1, keepdims=True)
    zs = logits - m
    out_ref[...] = zs - jnp.log(jnp.sum(jnp.exp(zs), axis=1, keepdims=True))


def _post2(accm, accd, h1, b2, fc3_w, fc3_b, fc4_w, fc4_b, ln2_w, ln2_b,
           fcf_w, fcf_b):
    return pl.pallas_call(
        _post2_body,
        grid=(N // _BN,),
        in_specs=[
            pl.BlockSpec((NC, _BN, HC), lambda i: (0, i, 0)),
            pl.BlockSpec((NC, _BN, 16), lambda i: (0, i, 0)),
            pl.BlockSpec((_BN, HC), lambda i: (i, 0)),
            pl.BlockSpec((1, HC), lambda i: (0, 0)),
            pl.BlockSpec((HC, HC), lambda i: (0, 0)),
            pl.BlockSpec((1, HC), lambda i: (0, 0)),
            pl.BlockSpec((HC, HC), lambda i: (0, 0)),
            pl.BlockSpec((1, HC), lambda i: (0, 0)),
            pl.BlockSpec((1, HC), lambda i: (0, 0)),
            pl.BlockSpec((1, HC), lambda i: (0, 0)),
            pl.BlockSpec((HC, NCLS), lambda i: (0, 0)),
            pl.BlockSpec((1, NCLS), lambda i: (0, 0)),
        ],
        out_specs=[pl.BlockSpec((_BN, NCLS), lambda i: (i, 0))],
        out_shape=[jax.ShapeDtypeStruct((N, NCLS), jnp.float32)],
    )(accm, accd, h1, b2, fc3_w, fc3_b, fc4_w, fc4_b, ln2_w, ln2_b,
      fcf_w, fcf_b)


def kernel(x, edge_index, W1, att_src1, att_dst1, b1, ln1_w, ln1_b, fc1_w,
           fc1_b, fc2_w, fc2_b, W2, att_src2, att_dst2, b2, fc3_w, fc3_b,
           fc4_w, fc4_b, ln2_w, ln2_b, fcf_w, fcf_b):
    src3 = edge_index[0].reshape(NW, EPC, CH)
    dst3 = edge_index[1].reshape(NW, EPC, CH)
    z64 = jnp.zeros((STRIPE, HC), jnp.float32)
    z16 = jnp.zeros((STRIPE, 16), jnp.float32)
    r1 = lambda v: v.reshape(1, -1)

    xpp1, tsl1, td1, m1 = _prep1(x, W1, att_src1, att_dst1)
    accm1, accd1 = _sc_layer(xpp1, tsl1, td1, src3, dst3, m1.reshape(16),
                             z64, z16)
    h1, xp2, tsl2, td2, m2 = _post1(
        accm1, accd1, r1(b1), fc1_w, r1(fc1_b), fc2_w, r1(fc2_b),
        r1(ln1_w), r1(ln1_b), W2, att_src2, att_dst2)
    accm2, accd2 = _sc_layer(xp2, tsl2, td2, src3, dst3, m2.reshape(16),
                             z64, z16)
    out = _post2(
        accm2, accd2, h1, r1(b2), fc3_w, r1(fc3_b), fc4_w,
        r1(fc4_b), r1(ln2_w), r1(ln2_b), fcf_w, r1(fcf_b))
    return out[0]


# speedup vs baseline: 1.5919x; 1.5919x over previous
import functools

import jax
import jax.numpy as jnp
import numpy as np
from jax import lax
from jax.experimental import pallas as pl
from jax.experimental.pallas import tpu as pltpu
from jax.experimental.pallas import tpu_sc as plsc

N = 10000
E = 320000
D = 128
H1 = 8
C1 = 8
HC = 64
NCLS = 40
W = HC + 16

NC = 2
NS = 16
NW = NC * NS
EPW = E // NW
CH = 125
EPC = EPW // CH
NP = 12000
STRIPE = NP // NS

_BN = 2000

_SC_PARAMS = pltpu.CompilerParams(use_tc_tiling_on_sc=False)


def _lrelu(x):
    return jnp.where(x >= 0, x, 0.2 * x)


def _prep1_body(x_ref, w1_ref, asrc_ref, adst_ref, comb_ref, td_ref,
                m1_ref, mx_ref):
    i = pl.program_id(0)
    ng = pl.num_programs(0)
    xp = jnp.dot(x_ref[...], w1_ref[...], preferred_element_type=jnp.float32)
    xph = xp.reshape(_BN, H1, C1)
    xp_perm = jnp.transpose(xph, (0, 2, 1)).reshape(_BN, HC)
    a_s = jnp.sum(xph * asrc_ref[...][None, :, :], axis=-1)
    a_d = jnp.sum(xph * adst_ref[...][None, :, :], axis=-1)
    comb_ref[...] = jnp.concatenate([xp_perm, a_s, a_s], axis=1)
    td_ref[...] = jnp.concatenate([a_d, a_d], axis=1)
    blkmax = jnp.concatenate(
        [jnp.max(a_s, axis=0), jnp.max(a_d, axis=0)])[None, :]

    @pl.when(i == 0)
    def _():
        mx_ref[...] = blkmax

    @pl.when(i > 0)
    def _():
        mx_ref[...] = jnp.maximum(mx_ref[...], blkmax)

    @pl.when(i == ng - 1)
    def _():
        mx = mx_ref[...]
        m = _lrelu(mx[:, 0:H1] + mx[:, H1:2 * H1])
        m1_ref[...] = jnp.concatenate([m, m], axis=1)


def _prep1(x, W1, att_src1, att_dst1):
    return pl.pallas_call(
        _prep1_body,
        grid=(N // _BN,),
        in_specs=[
            pl.BlockSpec((_BN, D), lambda i: (i, 0)),
            pl.BlockSpec((D, HC), lambda i: (0, 0)),
            pl.BlockSpec((H1, C1), lambda i: (0, 0)),
            pl.BlockSpec((H1, C1), lambda i: (0, 0)),
        ],
        out_specs=[
            pl.BlockSpec((_BN, W), lambda i: (i, 0)),
            pl.BlockSpec((_BN, 16), lambda i: (i, 0)),
            pl.BlockSpec((1, 16), lambda i: (0, 0)),
        ],
        out_shape=[
            jax.ShapeDtypeStruct((N, W), jnp.float32),
            jax.ShapeDtypeStruct((N, 16), jnp.float32),
            jax.ShapeDtypeStruct((1, 16), jnp.float32),
        ],
        scratch_shapes=[pltpu.VMEM((1, 16), jnp.float32)],
    )(x, W1, att_src1, att_dst1)


def _sc_layer(comb, td, src3, dst3, m, zrow):
    mesh = plsc.VectorSubcoreMesh(core_axis_name="c", subcore_axis_name="s")

    @functools.partial(
        pl.kernel,
        out_type=jax.ShapeDtypeStruct((NC, NP, W), jnp.float32),
        mesh=mesh,
        compiler_params=_SC_PARAMS,
        scratch_types=[
            pltpu.VMEM((EPC, CH), jnp.int32),
            pltpu.VMEM((EPC, CH), jnp.int32),
            pltpu.VMEM((16,), jnp.float32),
            pltpu.VMEM((2, CH, W), jnp.float32),
            pltpu.VMEM((2, CH, 16), jnp.float32),
            pltpu.VMEM((2, CH, W), jnp.float32),
            pltpu.VMEM_SHARED((NP, W), jnp.float32),
            pltpu.SemaphoreType.DMA((2,)),
            pltpu.SemaphoreType.DMA((2,)),
        ],
    )
    def k(comb_hbm, td_hbm, src_hbm, dst_hbm, m_hbm, z_hbm, acc_out,
          src_v, dst_v, m_v, sb, db, msgb, spm, gsem, ssem):
        c = lax.axis_index("c")
        s = lax.axis_index("s")
        wid = s * NC + c
        pltpu.sync_copy(z_hbm, spm.at[pl.ds(s * STRIPE, STRIPE)])
        pltpu.sync_copy(src_hbm.at[wid], src_v)
        pltpu.sync_copy(dst_hbm.at[wid], dst_v)
        pltpu.sync_copy(m_hbm, m_v)
        plsc.subcore_barrier()

        def fire(j, b):
            pltpu.make_async_copy(
                comb_hbm.at[src_v.at[j]], sb.at[b], gsem.at[b]).start()
            pltpu.make_async_copy(
                td_hbm.at[dst_v.at[j]], db.at[b], gsem.at[b]).start()

        def wait_g(b):
            pltpu.make_async_copy(
                comb_hbm.at[src_v.at[0]], sb.at[b], gsem.at[b]).wait()
            pltpu.make_async_copy(
                td_hbm.at[src_v.at[0]], db.at[b], gsem.at[b]).wait()

        def work(j, b):
            mv = m_v[...]
            sbb = sb.at[b]
            dbb = db.at[b]
            mbb = msgb.at[b]

            @pl.loop(0, CH)
            def _(e):
                z = sbb[e, pl.ds(HC, 16)] + dbb[e, :]
                ex = jnp.exp(_lrelu(z) - mv)
                mbb[e, pl.ds(HC, 16)] = ex
                for g in range(4):
                    mbb[e, pl.ds(16 * g, 16)] = sbb[e, pl.ds(16 * g, 16)] * ex

            pltpu.async_copy(mbb, spm.at[dst_v.at[j]], ssem.at[b], add=True)

        def wait_s(b):
            pltpu.make_async_copy(
                msgb.at[b], spm.at[dst_v.at[0]], ssem.at[b]).wait()

        fire(0, 0)

        @pl.loop(0, EPC, step=2)
        def _(j):
            @pl.when(j + 1 < EPC)
            def _():
                fire(j + 1, 1)
            wait_g(0)

            @pl.when(j >= 2)
            def _():
                wait_s(0)
            work(j, 0)

            @pl.when(j + 2 < EPC)
            def _():
                fire(j + 2, 0)
            wait_g(1)

            @pl.when(j >= 2)
            def _():
                wait_s(1)
            work(j + 1, 1)

        wait_s(0)
        wait_s(1)
        plsc.subcore_barrier()
        pltpu.sync_copy(spm.at[pl.ds(s * STRIPE, STRIPE)],
                        acc_out.at[c, pl.ds(s * STRIPE, STRIPE)])

    return k(comb, td, src3, dst3, m, zrow)


def _ln(h, w, b):
    mu = jnp.mean(h, axis=-1, keepdims=True)
    var = jnp.mean((h - mu) ** 2, axis=-1, keepdims=True)
    return (h - mu) / jnp.sqrt(var + 1e-05) * w + b


def _post1_body(acc0_ref, acc1_ref, b1_ref, fc1w_ref, fc1b_ref, fc2w_ref,
                fc2b_ref, ln1w_ref, ln1b_ref, w2_ref, as2_ref, ad2_ref,
                h1_ref, comb2_ref, td2_ref, m2_ref, mx_ref):
    i = pl.program_id(0)
    ng = pl.num_programs(0)
    acc = acc0_ref[0] + acc1_ref[0]
    den = acc[:, HC:HC + H1]
    nperm = acc[:, 0:HC].reshape(_BN, C1, H1)
    g = jnp.transpose(nperm, (0, 2, 1)) / (den[:, :, None] + 1e-16)
    h = jax.nn.relu(g.reshape(_BN, HC) + b1_ref[...])
    hc = h
    h = jax.nn.relu(
        jnp.dot(h, fc1w_ref[...], preferred_element_type=jnp.float32)
        + fc1b_ref[...])
    h = jax.nn.relu(
        jnp.dot(h, fc2w_ref[...], preferred_element_type=jnp.float32)
        + fc2b_ref[...] + hc)
    h = _ln(h, ln1w_ref[...], ln1b_ref[...])
    h1_ref[...] = h
    xp2 = jnp.dot(h, w2_ref[...], preferred_element_type=jnp.float32)
    a_s2 = jnp.sum(xp2 * as2_ref[...], axis=1, keepdims=True)
    a_d2 = jnp.sum(xp2 * ad2_ref[...], axis=1, keepdims=True)
    comb2_ref[...] = jnp.concatenate(
        [xp2, jnp.broadcast_to(a_s2, (_BN, 16))], axis=1)
    td2_ref[...] = jnp.broadcast_to(a_d2, (_BN, 16))
    blkmax = jnp.concatenate(
        [jnp.max(a_s2, axis=0), jnp.max(a_d2, axis=0),
         jnp.zeros((14,), jnp.float32)])[None, :]

    @pl.when(i == 0)
    def _():
        mx_ref[...] = blkmax

    @pl.when(i > 0)
    def _():
        mx_ref[...] = jnp.maximum(mx_ref[...], blkmax)

    @pl.when(i == ng - 1)
    def _():
        mx = mx_ref[...]
        m = _lrelu(mx[:, 0:1] + mx[:, 1:2])
        m2_ref[...] = jnp.broadcast_to(m, (1, 16))


def _post1(acc, b1, fc1_w, fc1_b, fc2_w, fc2_b, ln1_w, ln1_b, W2,
           att_src2, att_dst2):
    return pl.pallas_call(
        _post1_body,
        grid=(N // _BN,),
        in_specs=[
            pl.BlockSpec((1, _BN, W), lambda i: (0, i, 0)),
            pl.BlockSpec((1, _BN, W), lambda i: (1, i, 0)),
            pl.BlockSpec((1, HC), lambda i: (0, 0)),
            pl.BlockSpec((HC, HC), lambda i: (0, 0)),
            pl.BlockSpec((1, HC), lambda i: (0, 0)),
            pl.BlockSpec((HC, HC), lambda i: (0, 0)),
            pl.BlockSpec((1, HC), lambda i: (0, 0)),
            pl.BlockSpec((1, HC), lambda i: (0, 0)),
            pl.BlockSpec((1, HC), lambda i: (0, 0)),
            pl.BlockSpec((HC, HC), lambda i: (0, 0)),
            pl.BlockSpec((1, HC), lambda i: (0, 0)),
            pl.BlockSpec((1, HC), lambda i: (0, 0)),
        ],
        out_specs=[
            pl.BlockSpec((_BN, HC), lambda i: (i, 0)),
            pl.BlockSpec((_BN, W), lambda i: (i, 0)),
            pl.BlockSpec((_BN, 16), lambda i: (i, 0)),
            pl.BlockSpec((1, 16), lambda i: (0, 0)),
        ],
        out_shape=[
            jax.ShapeDtypeStruct((N, HC), jnp.float32),
            jax.ShapeDtypeStruct((N, W), jnp.float32),
            jax.ShapeDtypeStruct((N, 16), jnp.float32),
            jax.ShapeDtypeStruct((1, 16), jnp.float32),
        ],
        scratch_shapes=[pltpu.VMEM((1, 16), jnp.float32)],
    )(acc, acc, b1, fc1_w, fc1_b, fc2_w, fc2_b, ln1_w, ln1_b, W2,
      att_src2, att_dst2)


def _post2_body(acc0_ref, acc1_ref, h1_ref, b2_ref, fc3w_ref, fc3b_ref,
                fc4w_ref, fc4b_ref, ln2w_ref, ln2b_ref, fcfw_ref, fcfb_ref,
                out_ref):
    acc = acc0_ref[0] + acc1_ref[0]
    den = acc[:, HC:HC + 1]
    h = acc[:, 0:HC] / (den + 1e-16) + b2_ref[...]
    h = jax.nn.relu(
        jnp.dot(h, fc3w_ref[...], preferred_element_type=jnp.float32)
        + fc3b_ref[...])
    h = (jnp.dot(h, fc4w_ref[...], preferred_element_type=jnp.float32)
         + fc4b_ref[...] + h1_ref[...])
    h = _ln(h, ln2w_ref[...], ln2b_ref[...])
    logits = (jnp.dot(h, fcfw_ref[...], preferred_element_type=jnp.float32)
              + fcfb_ref[...])
    m = jnp.max(logits, axis=1, keepdims=True)
    zs = logits - m
    out_ref[...] = zs - jnp.log(jnp.sum(jnp.exp(zs), axis=1, keepdims=True))


def _post2(acc, h1, b2, fc3_w, fc3_b, fc4_w, fc4_b, ln2_w, ln2_b,
           fcf_w, fcf_b):
    return pl.pallas_call(
        _post2_body,
        grid=(N // _BN,),
        in_specs=[
            pl.BlockSpec((1, _BN, W), lambda i: (0, i, 0)),
            pl.BlockSpec((1, _BN, W), lambda i: (1, i, 0)),
            pl.BlockSpec((_BN, HC), lambda i: (i, 0)),
            pl.BlockSpec((1, HC), lambda i: (0, 0)),
            pl.BlockSpec((HC, HC), lambda i: (0, 0)),
            pl.BlockSpec((1, HC), lambda i: (0, 0)),
            pl.BlockSpec((HC, HC), lambda i: (0, 0)),
            pl.BlockSpec((1, HC), lambda i: (0, 0)),
            pl.BlockSpec((1, HC), lambda i: (0, 0)),
            pl.BlockSpec((1, HC), lambda i: (0, 0)),
            pl.BlockSpec((HC, NCLS), lambda i: (0, 0)),
            pl.BlockSpec((1, NCLS), lambda i: (0, 0)),
        ],
        out_specs=[pl.BlockSpec((_BN, NCLS), lambda i: (i, 0))],
        out_shape=[jax.ShapeDtypeStruct((N, NCLS), jnp.float32)],
    )(acc, acc, h1, b2, fc3_w, fc3_b, fc4_w, fc4_b, ln2_w, ln2_b,
      fcf_w, fcf_b)


def kernel(x, edge_index, W1, att_src1, att_dst1, b1, ln1_w, ln1_b, fc1_w,
           fc1_b, fc2_w, fc2_b, W2, att_src2, att_dst2, b2, fc3_w, fc3_b,
           fc4_w, fc4_b, ln2_w, ln2_b, fcf_w, fcf_b):
    src3 = edge_index[0].reshape(NW, EPC, CH)
    dst3 = edge_index[1].reshape(NW, EPC, CH)
    zrow = jnp.zeros((STRIPE, W), jnp.float32)
    r1 = lambda v: v.reshape(1, -1)

    comb1, td1, m1 = _prep1(x, W1, att_src1, att_dst1)
    acc1 = _sc_layer(comb1, td1, src3, dst3, m1.reshape(16), zrow)
    h1, comb2, td2, m2 = _post1(
        acc1, r1(b1), fc1_w, r1(fc1_b), fc2_w, r1(fc2_b),
        r1(ln1_w), r1(ln1_b), W2, att_src2, att_dst2)
    acc2 = _sc_layer(comb2, td2, src3, dst3, m2.reshape(16), zrow)
    out = _post2(
        acc2, h1, r1(b2), fc3_w, r1(fc3_b), fc4_w,
        r1(fc4_b), r1(ln2_w), r1(ln2_b), fcf_w, r1(fcf_b))
    return out[0]


# speedup vs baseline: 1.5997x; 1.0049x over previous
import functools

import jax
import jax.numpy as jnp
import numpy as np
from jax import lax
from jax.experimental import pallas as pl
from jax.experimental.pallas import tpu as pltpu
from jax.experimental.pallas import tpu_sc as plsc

N = 10000
E = 320000
D = 128
H1 = 8
C1 = 8
HC = 64
NCLS = 40
W = HC + 16

NC = 2
NS = 16
NW = NC * NS
EPW = E // NW
CH = 80
EPC = EPW // CH
NP = 12000
STRIPE = NP // NS

_BN = 2000

_SC_PARAMS = pltpu.CompilerParams(use_tc_tiling_on_sc=False)


def _lrelu(x):
    return jnp.where(x >= 0, x, 0.2 * x)


def _prep1_body(x_ref, w1_ref, asrc_ref, adst_ref, comb_ref, td_ref,
                m1_ref, mx_ref):
    i = pl.program_id(0)
    ng = pl.num_programs(0)
    xp = jnp.dot(x_ref[...], w1_ref[...], preferred_element_type=jnp.float32)
    xph = xp.reshape(_BN, H1, C1)
    xp_perm = jnp.transpose(xph, (0, 2, 1)).reshape(_BN, HC)
    a_s = jnp.sum(xph * asrc_ref[...][None, :, :], axis=-1)
    a_d = jnp.sum(xph * adst_ref[...][None, :, :], axis=-1)
    comb_ref[...] = jnp.concatenate([xp_perm, a_s, a_s], axis=1)
    td_ref[...] = jnp.concatenate([a_d, a_d], axis=1)
    blkmax = jnp.concatenate(
        [jnp.max(a_s, axis=0), jnp.max(a_d, axis=0)])[None, :]

    @pl.when(i == 0)
    def _():
        mx_ref[...] = blkmax

    @pl.when(i > 0)
    def _():
        mx_ref[...] = jnp.maximum(mx_ref[...], blkmax)

    @pl.when(i == ng - 1)
    def _():
        mx = mx_ref[...]
        m = _lrelu(mx[:, 0:H1] + mx[:, H1:2 * H1])
        m1_ref[...] = jnp.concatenate([m, m], axis=1)


def _prep1(x, W1, att_src1, att_dst1):
    return pl.pallas_call(
        _prep1_body,
        grid=(N // _BN,),
        in_specs=[
            pl.BlockSpec((_BN, D), lambda i: (i, 0)),
            pl.BlockSpec((D, HC), lambda i: (0, 0)),
            pl.BlockSpec((H1, C1), lambda i: (0, 0)),
            pl.BlockSpec((H1, C1), lambda i: (0, 0)),
        ],
        out_specs=[
            pl.BlockSpec((_BN, W), lambda i: (i, 0)),
            pl.BlockSpec((_BN, 16), lambda i: (i, 0)),
            pl.BlockSpec((1, 16), lambda i: (0, 0)),
        ],
        out_shape=[
            jax.ShapeDtypeStruct((N, W), jnp.float32),
            jax.ShapeDtypeStruct((N, 16), jnp.float32),
            jax.ShapeDtypeStruct((1, 16), jnp.float32),
        ],
        scratch_shapes=[pltpu.VMEM((1, 16), jnp.float32)],
    )(x, W1, att_src1, att_dst1)


def _sc_layer(comb, td, src3, dst3, m, zrow):
    mesh = plsc.VectorSubcoreMesh(core_axis_name="c", subcore_axis_name="s")

    @functools.partial(
        pl.kernel,
        out_type=jax.ShapeDtypeStruct((NC, NP, W), jnp.float32),
        mesh=mesh,
        compiler_params=_SC_PARAMS,
        scratch_types=[
            pltpu.VMEM((EPC, CH), jnp.int32),
            pltpu.VMEM((EPC, CH), jnp.int32),
            pltpu.VMEM((16,), jnp.float32),
            pltpu.VMEM((4, CH, W), jnp.float32),
            pltpu.VMEM((4, CH, 16), jnp.float32),
            pltpu.VMEM((2, CH, W), jnp.float32),
            pltpu.VMEM_SHARED((NP, W), jnp.float32),
            pltpu.SemaphoreType.DMA((4,)),
            pltpu.SemaphoreType.DMA((2,)),
        ],
    )
    def k(comb_hbm, td_hbm, src_hbm, dst_hbm, m_hbm, z_hbm, acc_out,
          src_v, dst_v, m_v, sb, db, msgb, spm, gsem, ssem):
        c = lax.axis_index("c")
        s = lax.axis_index("s")
        wid = s * NC + c
        pltpu.sync_copy(z_hbm, spm.at[pl.ds(s * STRIPE, STRIPE)])
        pltpu.sync_copy(src_hbm.at[wid], src_v)
        pltpu.sync_copy(dst_hbm.at[wid], dst_v)
        pltpu.sync_copy(m_hbm, m_v)
        plsc.subcore_barrier()

        def fire(j, b):
            pltpu.make_async_copy(
                comb_hbm.at[src_v.at[j]], sb.at[b], gsem.at[b]).start()
            pltpu.make_async_copy(
                td_hbm.at[dst_v.at[j]], db.at[b], gsem.at[b]).start()

        def wait_g(b):
            pltpu.make_async_copy(
                comb_hbm.at[src_v.at[0]], sb.at[b], gsem.at[b]).wait()
            pltpu.make_async_copy(
                td_hbm.at[src_v.at[0]], db.at[b], gsem.at[b]).wait()

        def work(j, b, sbuf):
            mv = m_v[...]
            sbb = sb.at[b]
            dbb = db.at[b]
            mbb = msgb.at[sbuf]

            @pl.loop(0, CH)
            def _(e):
                z = sbb[e, pl.ds(HC, 16)] + dbb[e, :]
                ex = jnp.exp(_lrelu(z) - mv)
                mbb[e, pl.ds(HC, 16)] = ex
                for g in range(4):
                    mbb[e, pl.ds(16 * g, 16)] = sbb[e, pl.ds(16 * g, 16)] * ex

            pltpu.async_copy(mbb, spm.at[dst_v.at[j]], ssem.at[sbuf],
                             add=True)

        def wait_s(sbuf):
            pltpu.make_async_copy(
                msgb.at[sbuf], spm.at[dst_v.at[0]], ssem.at[sbuf]).wait()

        fire(0, 0)
        fire(1, 1)
        fire(2, 2)

        @pl.loop(0, EPC - 1, step=4)
        def _(j):
            for u in range(4):
                q = j + u

                @pl.when(q + 3 < EPC)
                def _():
                    fire(q + 3, (u + 3) % 4)
                wait_g(u)

                @pl.when(q >= 2)
                def _():
                    wait_s(u % 2)
                work(q, u, u % 2)

        wait_g(0)
        wait_s(0)
        work(EPC - 1, 0, 0)
        wait_s(1)
        wait_s(0)
        plsc.subcore_barrier()
        pltpu.sync_copy(spm.at[pl.ds(s * STRIPE, STRIPE)],
                        acc_out.at[c, pl.ds(s * STRIPE, STRIPE)])

    return k(comb, td, src3, dst3, m, zrow)


def _ln(h, w, b):
    mu = jnp.mean(h, axis=-1, keepdims=True)
    var = jnp.mean((h - mu) ** 2, axis=-1, keepdims=True)
    return (h - mu) / jnp.sqrt(var + 1e-05) * w + b


def _post1_body(acc0_ref, acc1_ref, b1_ref, fc1w_ref, fc1b_ref, fc2w_ref,
                fc2b_ref, ln1w_ref, ln1b_ref, w2_ref, as2_ref, ad2_ref,
                h1_ref, comb2_ref, td2_ref, m2_ref, mx_ref):
    i = pl.program_id(0)
    ng = pl.num_programs(0)
    acc = acc0_ref[0] + acc1_ref[0]
    den = acc[:, HC:HC + H1]
    nperm = acc[:, 0:HC].reshape(_BN, C1, H1)
    g = jnp.transpose(nperm, (0, 2, 1)) / (den[:, :, None] + 1e-16)
    h = jax.nn.relu(g.reshape(_BN, HC) + b1_ref[...])
    hc = h
    h = jax.nn.relu(
        jnp.dot(h, fc1w_ref[...], preferred_element_type=jnp.float32)
        + fc1b_ref[...])
    h = jax.nn.relu(
        jnp.dot(h, fc2w_ref[...], preferred_element_type=jnp.float32)
        + fc2b_ref[...] + hc)
    h = _ln(h, ln1w_ref[...], ln1b_ref[...])
    h1_ref[...] = h
    xp2 = jnp.dot(h, w2_ref[...], preferred_element_type=jnp.float32)
    a_s2 = jnp.sum(xp2 * as2_ref[...], axis=1, keepdims=True)
    a_d2 = jnp.sum(xp2 * ad2_ref[...], axis=1, keepdims=True)
    comb2_ref[...] = jnp.concatenate(
        [xp2, jnp.broadcast_to(a_s2, (_BN, 16))], axis=1)
    td2_ref[...] = jnp.broadcast_to(a_d2, (_BN, 16))
    blkmax = jnp.concatenate(
        [jnp.max(a_s2, axis=0), jnp.max(a_d2, axis=0),
         jnp.zeros((14,), jnp.float32)])[None, :]

    @pl.when(i == 0)
    def _():
        mx_ref[...] = blkmax

    @pl.when(i > 0)
    def _():
        mx_ref[...] = jnp.maximum(mx_ref[...], blkmax)

    @pl.when(i == ng - 1)
    def _():
        mx = mx_ref[...]
        m = _lrelu(mx[:, 0:1] + mx[:, 1:2])
        m2_ref[...] = jnp.broadcast_to(m, (1, 16))


def _post1(acc, b1, fc1_w, fc1_b, fc2_w, fc2_b, ln1_w, ln1_b, W2,
           att_src2, att_dst2):
    return pl.pallas_call(
        _post1_body,
        grid=(N // _BN,),
        in_specs=[
            pl.BlockSpec((1, _BN, W), lambda i: (0, i, 0)),
            pl.BlockSpec((1, _BN, W), lambda i: (1, i, 0)),
            pl.BlockSpec((1, HC), lambda i: (0, 0)),
            pl.BlockSpec((HC, HC), lambda i: (0, 0)),
            pl.BlockSpec((1, HC), lambda i: (0, 0)),
            pl.BlockSpec((HC, HC), lambda i: (0, 0)),
            pl.BlockSpec((1, HC), lambda i: (0, 0)),
            pl.BlockSpec((1, HC), lambda i: (0, 0)),
            pl.BlockSpec((1, HC), lambda i: (0, 0)),
            pl.BlockSpec((HC, HC), lambda i: (0, 0)),
            pl.BlockSpec((1, HC), lambda i: (0, 0)),
            pl.BlockSpec((1, HC), lambda i: (0, 0)),
        ],
        out_specs=[
            pl.BlockSpec((_BN, HC), lambda i: (i, 0)),
            pl.BlockSpec((_BN, W), lambda i: (i, 0)),
            pl.BlockSpec((_BN, 16), lambda i: (i, 0)),
            pl.BlockSpec((1, 16), lambda i: (0, 0)),
        ],
        out_shape=[
            jax.ShapeDtypeStruct((N, HC), jnp.float32),
            jax.ShapeDtypeStruct((N, W), jnp.float32),
            jax.ShapeDtypeStruct((N, 16), jnp.float32),
            jax.ShapeDtypeStruct((1, 16), jnp.float32),
        ],
        scratch_shapes=[pltpu.VMEM((1, 16), jnp.float32)],
    )(acc, acc, b1, fc1_w, fc1_b, fc2_w, fc2_b, ln1_w, ln1_b, W2,
      att_src2, att_dst2)


def _post2_body(acc0_ref, acc1_ref, h1_ref, b2_ref, fc3w_ref, fc3b_ref,
                fc4w_ref, fc4b_ref, ln2w_ref, ln2b_ref, fcfw_ref, fcfb_ref,
                out_ref):
    acc = acc0_ref[0] + acc1_ref[0]
    den = acc[:, HC:HC + 1]
    h = acc[:, 0:HC] / (den + 1e-16) + b2_ref[...]
    h = jax.nn.relu(
        jnp.dot(h, fc3w_ref[...], preferred_element_type=jnp.float32)
        + fc3b_ref[...])
    h = (jnp.dot(h, fc4w_ref[...], preferred_element_type=jnp.float32)
         + fc4b_ref[...] + h1_ref[...])
    h = _ln(h, ln2w_ref[...], ln2b_ref[...])
    logits = (jnp.dot(h, fcfw_ref[...], preferred_element_type=jnp.float32)
              + fcfb_ref[...])
    m = jnp.max(logits, axis=1, keepdims=True)
    zs = logits - m
    out_ref[...] = zs - jnp.log(jnp.sum(jnp.exp(zs), axis=1, keepdims=True))


def _post2(acc, h1, b2, fc3_w, fc3_b, fc4_w, fc4_b, ln2_w, ln2_b,
           fcf_w, fcf_b):
    return pl.pallas_call(
        _post2_body,
        grid=(N // _BN,),
        in_specs=[
            pl.BlockSpec((1, _BN, W), lambda i: (0, i, 0)),
            pl.BlockSpec((1, _BN, W), lambda i: (1, i, 0)),
            pl.BlockSpec((_BN, HC), lambda i: (i, 0)),
            pl.BlockSpec((1, HC), lambda i: (0, 0)),
            pl.BlockSpec((HC, HC), lambda i: (0, 0)),
            pl.BlockSpec((1, HC), lambda i: (0, 0)),
            pl.BlockSpec((HC, HC), lambda i: (0, 0)),
            pl.BlockSpec((1, HC), lambda i: (0, 0)),
            pl.BlockSpec((1, HC), lambda i: (0, 0)),
            pl.BlockSpec((1, HC), lambda i: (0, 0)),
            pl.BlockSpec((HC, NCLS), lambda i: (0, 0)),
            pl.BlockSpec((1, NCLS), lambda i: (0, 0)),
        ],
        out_specs=[pl.BlockSpec((_BN, NCLS), lambda i: (i, 0))],
        out_shape=[jax.ShapeDtypeStruct((N, NCLS), jnp.float32)],
    )(acc, acc, h1, b2, fc3_w, fc3_b, fc4_w, fc4_b, ln2_w, ln2_b,
      fcf_w, fcf_b)


def kernel(x, edge_index, W1, att_src1, att_dst1, b1, ln1_w, ln1_b, fc1_w,
           fc1_b, fc2_w, fc2_b, W2, att_src2, att_dst2, b2, fc3_w, fc3_b,
           fc4_w, fc4_b, ln2_w, ln2_b, fcf_w, fcf_b):
    src3 = edge_index[0].reshape(NW, EPC, CH)
    dst3 = edge_index[1].reshape(NW, EPC, CH)
    zrow = jnp.zeros((STRIPE, W), jnp.float32)
    r1 = lambda v: v.reshape(1, -1)

    comb1, td1, m1 = _prep1(x, W1, att_src1, att_dst1)
    acc1 = _sc_layer(comb1, td1, src3, dst3, m1.reshape(16), zrow)
    h1, comb2, td2, m2 = _post1(
        acc1, r1(b1), fc1_w, r1(fc1_b), fc2_w, r1(fc2_b),
        r1(ln1_w), r1(ln1_b), W2, att_src2, att_dst2)
    acc2 = _sc_layer(comb2, td2, src3, dst3, m2.reshape(16), zrow)
    out = _post2(
        acc2, h1, r1(b2), fc3_w, r1(fc3_b), fc4_w,
        r1(fc4_b), r1(ln2_w), r1(ln2_b), fcf_w, r1(fcf_b))
    return out[0]


# speedup vs baseline: 1.6010x; 1.0008x over previous
import functools

import jax
import jax.numpy as jnp
from jax import lax
from jax.experimental import pallas as pl
from jax.experimental.pallas import tpu as pltpu
from jax.experimental.pallas import tpu_sc as plsc

N = 10000
E = 320000
D = 128
H1 = 8
C1 = 8
HC = 64
NCLS = 40
W = HC + 16

NC = 2
NS = 16
NW = NC * NS
EPW = E // NW
CH = 80
EPC = EPW // CH
NP = 12000
STRIPE = NP // NS

_BN = 2000

_SC_PARAMS = pltpu.CompilerParams(use_tc_tiling_on_sc=False)


def _lrelu(x):
    return jnp.where(x >= 0, x, 0.2 * x)


def _prep1_body(x_ref, w1_ref, asrc_ref, adst_ref, comb_ref, td_ref,
                m1_ref, mx_ref):
    i = pl.program_id(0)
    ng = pl.num_programs(0)
    xp = jnp.dot(x_ref[...], w1_ref[...], preferred_element_type=jnp.float32)
    xph = xp.reshape(_BN, H1, C1)
    xp_perm = jnp.transpose(xph, (0, 2, 1)).reshape(_BN, HC)
    a_s = jnp.sum(xph * asrc_ref[...][None, :, :], axis=-1)
    a_d = jnp.sum(xph * adst_ref[...][None, :, :], axis=-1)
    comb_ref[...] = jnp.concatenate([xp_perm, a_s, a_s], axis=1)
    td_ref[...] = jnp.concatenate([a_d, a_d], axis=1)
    blkmax = jnp.concatenate(
        [jnp.max(a_s, axis=0), jnp.max(a_d, axis=0)])[None, :]

    @pl.when(i == 0)
    def _():
        mx_ref[...] = blkmax

    @pl.when(i > 0)
    def _():
        mx_ref[...] = jnp.maximum(mx_ref[...], blkmax)

    @pl.when(i == ng - 1)
    def _():
        mx = mx_ref[...]
        m = _lrelu(mx[:, 0:H1] + mx[:, H1:2 * H1])
        m1_ref[...] = jnp.concatenate([m, m], axis=1)


def _prep1(x, W1, att_src1, att_dst1):
    return pl.pallas_call(
        _prep1_body,
        grid=(N // _BN,),
        in_specs=[
            pl.BlockSpec((_BN, D), lambda i: (i, 0)),
            pl.BlockSpec((D, HC), lambda i: (0, 0)),
            pl.BlockSpec((H1, C1), lambda i: (0, 0)),
            pl.BlockSpec((H1, C1), lambda i: (0, 0)),
        ],
        out_specs=[
            pl.BlockSpec((_BN, W), lambda i: (i, 0)),
            pl.BlockSpec((_BN, 16), lambda i: (i, 0)),
            pl.BlockSpec((1, 16), lambda i: (0, 0)),
        ],
        out_shape=[
            jax.ShapeDtypeStruct((N, W), jnp.float32),
            jax.ShapeDtypeStruct((N, 16), jnp.float32),
            jax.ShapeDtypeStruct((1, 16), jnp.float32),
        ],
        scratch_shapes=[pltpu.VMEM((1, 16), jnp.float32)],
    )(x, W1, att_src1, att_dst1)


def _sc_layer(comb, td, src3, dst3, m, zrow):
    mesh = plsc.VectorSubcoreMesh(core_axis_name="c", subcore_axis_name="s")

    @functools.partial(
        pl.kernel,
        out_type=jax.ShapeDtypeStruct((NC, NP, W), jnp.float32),
        mesh=mesh,
        compiler_params=_SC_PARAMS,
        scratch_types=[
            pltpu.VMEM((EPC, CH), jnp.int32),
            pltpu.VMEM((EPC, CH), jnp.int32),
            pltpu.VMEM((16,), jnp.float32),
            pltpu.VMEM((4, CH, W), jnp.float32),
            pltpu.VMEM((4, CH, 16), jnp.float32),
            pltpu.VMEM((2, CH, W), jnp.float32),
            pltpu.VMEM_SHARED((NP, W), jnp.float32),
            pltpu.SemaphoreType.DMA((4,)),
            pltpu.SemaphoreType.DMA((2,)),
        ],
    )
    def k(comb_hbm, td_hbm, src_hbm, dst_hbm, m_hbm, z_hbm, acc_out,
          src_v, dst_v, m_v, sb, db, msgb, spm, gsem, ssem):
        c = lax.axis_index("c")
        s = lax.axis_index("s")
        wid = s * NC + c
        pltpu.sync_copy(z_hbm, spm.at[pl.ds(s * STRIPE, STRIPE)])
        pltpu.sync_copy(src_hbm.at[wid], src_v)
        pltpu.sync_copy(dst_hbm.at[wid], dst_v)
        pltpu.sync_copy(m_hbm, m_v)
        plsc.subcore_barrier()

        def fire(j, b):
            pltpu.make_async_copy(
                comb_hbm.at[src_v.at[j]], sb.at[b], gsem.at[b]).start()
            pltpu.make_async_copy(
                td_hbm.at[dst_v.at[j]], db.at[b], gsem.at[b]).start()

        def wait_g(b):
            pltpu.make_async_copy(
                comb_hbm.at[src_v.at[0]], sb.at[b], gsem.at[b]).wait()
            pltpu.make_async_copy(
                td_hbm.at[src_v.at[0]], db.at[b], gsem.at[b]).wait()

        def work(j, b, sbuf):
            mv = m_v[...]
            sbb = sb.at[b]
            dbb = db.at[b]
            mbb = msgb.at[sbuf]

            @pl.loop(0, CH)
            def _(e):
                z = sbb[e, pl.ds(HC, 16)] + dbb[e, :]
                ex = jnp.exp(_lrelu(z) - mv)
                mbb[e, pl.ds(HC, 16)] = ex
                for g in range(4):
                    mbb[e, pl.ds(16 * g, 16)] = sbb[e, pl.ds(16 * g, 16)] * ex

            pltpu.async_copy(mbb, spm.at[dst_v.at[j]], ssem.at[sbuf],
                             add=True)

        def wait_s(sbuf):
            pltpu.make_async_copy(
                msgb.at[sbuf], spm.at[dst_v.at[0]], ssem.at[sbuf]).wait()

        fire(0, 0)
        fire(1, 1)
        fire(2, 2)

        @pl.loop(0, EPC - 1, step=4)
        def _(j):
            for u in range(4):
                q = j + u

                @pl.when(q + 3 < EPC)
                def _():
                    fire(q + 3, (u + 3) % 4)
                wait_g(u)

                @pl.when(q >= 2)
                def _():
                    wait_s(u % 2)
                work(q, u, u % 2)

        wait_g(0)
        wait_s(0)
        work(EPC - 1, 0, 0)
        wait_s(1)
        wait_s(0)
        plsc.subcore_barrier()
        pltpu.sync_copy(spm.at[pl.ds(s * STRIPE, STRIPE)],
                        acc_out.at[c, pl.ds(s * STRIPE, STRIPE)])

    return k(comb, td, src3, dst3, m, zrow)


def _ln(h, w, b):
    mu = jnp.mean(h, axis=-1, keepdims=True)
    var = jnp.mean((h - mu) ** 2, axis=-1, keepdims=True)
    return (h - mu) / jnp.sqrt(var + 1e-05) * w + b


def _post1_body(acc0_ref, acc1_ref, b1_ref, fc1w_ref, fc1b_ref, fc2w_ref,
                fc2b_ref, ln1w_ref, ln1b_ref, w2_ref, as2_ref, ad2_ref,
                h1_ref, comb2_ref, td2_ref, m2_ref, mx_ref):
    i = pl.program_id(0)
    ng = pl.num_programs(0)
    acc = acc0_ref[0] + acc1_ref[0]
    den = acc[:, HC:HC + H1]
    nperm = acc[:, 0:HC].reshape(_BN, C1, H1)
    g = jnp.transpose(nperm, (0, 2, 1)) / (den[:, :, None] + 1e-16)
    h = jax.nn.relu(g.reshape(_BN, HC) + b1_ref[...])
    hc = h
    h = jax.nn.relu(
        jnp.dot(h, fc1w_ref[...], preferred_element_type=jnp.float32)
        + fc1b_ref[...])
    h = jax.nn.relu(
        jnp.dot(h, fc2w_ref[...], preferred_element_type=jnp.float32)
        + fc2b_ref[...] + hc)
    h = _ln(h, ln1w_ref[...], ln1b_ref[...])
    h1_ref[...] = h
    xp2 = jnp.dot(h, w2_ref[...], preferred_element_type=jnp.float32)
    a_s2 = jnp.sum(xp2 * as2_ref[...], axis=1, keepdims=True)
    a_d2 = jnp.sum(xp2 * ad2_ref[...], axis=1, keepdims=True)
    comb2_ref[...] = jnp.concatenate(
        [xp2, jnp.broadcast_to(a_s2, (_BN, 16))], axis=1)
    td2_ref[...] = jnp.broadcast_to(a_d2, (_BN, 16))
    blkmax = jnp.concatenate(
        [jnp.max(a_s2, axis=0), jnp.max(a_d2, axis=0),
         jnp.zeros((14,), jnp.float32)])[None, :]

    @pl.when(i == 0)
    def _():
        mx_ref[...] = blkmax

    @pl.when(i > 0)
    def _():
        mx_ref[...] = jnp.maximum(mx_ref[...], blkmax)

    @pl.when(i == ng - 1)
    def _():
        mx = mx_ref[...]
        m = _lrelu(mx[:, 0:1] + mx[:, 1:2])
        m2_ref[...] = jnp.broadcast_to(m, (1, 16))


def _post1(acc, b1, fc1_w, fc1_b, fc2_w, fc2_b, ln1_w, ln1_b, W2,
           att_src2, att_dst2):
    return pl.pallas_call(
        _post1_body,
        grid=(N // _BN,),
        in_specs=[
            pl.BlockSpec((1, _BN, W), lambda i: (0, i, 0)),
            pl.BlockSpec((1, _BN, W), lambda i: (1, i, 0)),
            pl.BlockSpec((1, HC), lambda i: (0, 0)),
            pl.BlockSpec((HC, HC), lambda i: (0, 0)),
            pl.BlockSpec((1, HC), lambda i: (0, 0)),
            pl.BlockSpec((HC, HC), lambda i: (0, 0)),
            pl.BlockSpec((1, HC), lambda i: (0, 0)),
            pl.BlockSpec((1, HC), lambda i: (0, 0)),
            pl.BlockSpec((1, HC), lambda i: (0, 0)),
            pl.BlockSpec((HC, HC), lambda i: (0, 0)),
            pl.BlockSpec((1, HC), lambda i: (0, 0)),
            pl.BlockSpec((1, HC), lambda i: (0, 0)),
        ],
        out_specs=[
            pl.BlockSpec((_BN, HC), lambda i: (i, 0)),
            pl.BlockSpec((_BN, W), lambda i: (i, 0)),
            pl.BlockSpec((_BN, 16), lambda i: (i, 0)),
            pl.BlockSpec((1, 16), lambda i: (0, 0)),
        ],
        out_shape=[
            jax.ShapeDtypeStruct((N, HC), jnp.float32),
            jax.ShapeDtypeStruct((N, W), jnp.float32),
            jax.ShapeDtypeStruct((N, 16), jnp.float32),
            jax.ShapeDtypeStruct((1, 16), jnp.float32),
        ],
        scratch_shapes=[pltpu.VMEM((1, 16), jnp.float32)],
    )(acc, acc, b1, fc1_w, fc1_b, fc2_w, fc2_b, ln1_w, ln1_b, W2,
      att_src2, att_dst2)


def _post2_body(acc0_ref, acc1_ref, h1_ref, b2_ref, fc3w_ref, fc3b_ref,
                fc4w_ref, fc4b_ref, ln2w_ref, ln2b_ref, fcfw_ref, fcfb_ref,
                out_ref):
    acc = acc0_ref[0] + acc1_ref[0]
    den = acc[:, HC:HC + 1]
    h = acc[:, 0:HC] / (den + 1e-16) + b2_ref[...]
    h = jax.nn.relu(
        jnp.dot(h, fc3w_ref[...], preferred_element_type=jnp.float32)
        + fc3b_ref[...])
    h = (jnp.dot(h, fc4w_ref[...], preferred_element_type=jnp.float32)
         + fc4b_ref[...] + h1_ref[...])
    h = _ln(h, ln2w_ref[...], ln2b_ref[...])
    logits = (jnp.dot(h, fcfw_ref[...], preferred_element_type=jnp.float32)
              + fcfb_ref[...])
    m = jnp.max(logits, axis=1, keepdims=True)
    zs = logits - m
    out_ref[...] = zs - jnp.log(jnp.sum(jnp.exp(zs), axis=1, keepdims=True))


def _post2(acc, h1, b2, fc3_w, fc3_b, fc4_w, fc4_b, ln2_w, ln2_b,
           fcf_w, fcf_b):
    return pl.pallas_call(
        _post2_body,
        grid=(N // _BN,),
        in_specs=[
            pl.BlockSpec((1, _BN, W), lambda i: (0, i, 0)),
            pl.BlockSpec((1, _BN, W), lambda i: (1, i, 0)),
            pl.BlockSpec((_BN, HC), lambda i: (i, 0)),
            pl.BlockSpec((1, HC), lambda i: (0, 0)),
            pl.BlockSpec((HC, HC), lambda i: (0, 0)),
            pl.BlockSpec((1, HC), lambda i: (0, 0)),
            pl.BlockSpec((HC, HC), lambda i: (0, 0)),
            pl.BlockSpec((1, HC), lambda i: (0, 0)),
            pl.BlockSpec((1, HC), lambda i: (0, 0)),
            pl.BlockSpec((1, HC), lambda i: (0, 0)),
            pl.BlockSpec((HC, NCLS), lambda i: (0, 0)),
            pl.BlockSpec((1, NCLS), lambda i: (0, 0)),
        ],
        out_specs=[pl.BlockSpec((_BN, NCLS), lambda i: (i, 0))],
        out_shape=[jax.ShapeDtypeStruct((N, NCLS), jnp.float32)],
    )(acc, acc, h1, b2, fc3_w, fc3_b, fc4_w, fc4_b, ln2_w, ln2_b,
      fcf_w, fcf_b)


def kernel(x, edge_index, W1, att_src1, att_dst1, b1, ln1_w, ln1_b, fc1_w,
           fc1_b, fc2_w, fc2_b, W2, att_src2, att_dst2, b2, fc3_w, fc3_b,
           fc4_w, fc4_b, ln2_w, ln2_b, fcf_w, fcf_b):
    src3 = edge_index[0].reshape(NW, EPC, CH)
    dst3 = edge_index[1].reshape(NW, EPC, CH)
    zrow = jnp.zeros((STRIPE, W), jnp.float32)
    r1 = lambda v: v.reshape(1, -1)

    comb1, td1, m1 = _prep1(x, W1, att_src1, att_dst1)
    acc1 = _sc_layer(comb1, td1, src3, dst3, m1.reshape(16), zrow)
    h1, comb2, td2, m2 = _post1(
        acc1, r1(b1), fc1_w, r1(fc1_b), fc2_w, r1(fc2_b),
        r1(ln1_w), r1(ln1_b), W2, att_src2, att_dst2)
    acc2 = _sc_layer(comb2, td2, src3, dst3, m2.reshape(16), zrow)
    out = _post2(
        acc2, h1, r1(b2), fc3_w, r1(fc3_b), fc4_w,
        r1(fc4_b), r1(ln2_w), r1(ln2_b), fcf_w, r1(fcf_b))
    return out[0]
